# Initial kernel scaffold; baseline (speedup 1.0000x reference)
import jax
import jax.numpy as jnp
from jax.experimental import pallas as pl


def kernel(head_node, objective_nodes, value_nodes, ei_vo_src, ei_vo_dst, ei_oh_src, ei_oh_dst, opponent_encoding, vo_Wl, vo_bl, vo_Wr, vo_br, vo_att, vo_b, oh_Wl, oh_bl, oh_Wr, oh_br, oh_att, oh_b, vf_W, vf_b):
    raise NotImplementedError("write your pallas kernel here")


# speedup vs baseline: 7.3330x; 7.3330x over previous
import functools

import jax
import jax.numpy as jnp
from jax import lax
from jax.experimental import pallas as pl
from jax.experimental.pallas import tpu as pltpu
from jax.experimental.pallas import tpu_sc as plsc

B = 4
NV = 50000
NO = 2000
E1 = 100000
E2 = 2000
H = 512
NA = 8
EA = E1 + NO
EP = 102400
NE = B * EP
NB = 2048
NW = 32
PER = NE // NW
CH = 3200
EBLK = 512
NBLK = NE // EBLK


def _leaky(x):
    return jnp.where(x > 0, x, 0.2 * x)


def _tc0_body(cat_ref, w_ref, b0_ref, att_ref, out_ref):
    outs = []
    for b in range(B):
        z = jnp.dot(cat_ref[b], w_ref[...],
                    preferred_element_type=jnp.float32) + b0_ref[...]
        c = jnp.dot(_leaky(z), att_ref[...],
                    preferred_element_type=jnp.float32)
        outs.append(c.reshape(1, NO))
    out_ref[...] = jnp.concatenate(outs, axis=0)


def _sc1_body(src_tab, dst_tab, sidx, didx, gs_out, gd_out,
              sbuf_i, dbuf_i, sbuf_r, dbuf_r, sem1, sem2):
    wid = lax.axis_index("s") * 2 + lax.axis_index("c")
    base = wid * PER
    for k in range(PER // CH):
        off = base + k * CH
        pltpu.sync_copy(sidx.at[pl.ds(off, CH)], sbuf_i)
        pltpu.sync_copy(didx.at[pl.ds(off, CH)], dbuf_i)
        cp1 = pltpu.async_copy(src_tab.at[sbuf_i], sbuf_r, sem1)
        cp2 = pltpu.async_copy(dst_tab.at[dbuf_i], dbuf_r, sem2)
        cp1.wait()
        cp2.wait()
        pltpu.sync_copy(sbuf_r, gs_out.at[pl.ds(off, CH)])
        pltpu.sync_copy(dbuf_r, gd_out.at[pl.ds(off, CH)])


def _tc1_body(gs_ref, gd_ref, sr_ref, dr_ref, ws_ref, wd_ref, b0_ref,
              att_ref, out_ref):
    i = pl.program_id(0)
    gs = gs_ref[...]
    gd = gd_ref[...]
    z = (jnp.dot(gs, ws_ref[...], preferred_element_type=jnp.float32)
         + jnp.dot(gd, wd_ref[...], preferred_element_type=jnp.float32)
         + b0_ref[...])
    e = jnp.dot(_leaky(z), att_ref[...],
                preferred_element_type=jnp.float32)
    srcv = sr_ref[...].reshape(EBLK, 1)
    dstv = dr_ref[...].reshape(EBLK, 1)
    pos = ((i % (EP // EBLK)) * EBLK
           + lax.broadcasted_iota(jnp.int32, (EBLK, 1), 0))
    bad = ((pos < E1) & (srcv == dstv)) | (pos >= EA)
    e = jnp.where(bad, -1e30, e)
    ex = jnp.exp(jnp.minimum(e - gd[:, 2:3], 70.0))
    out_ref[...] = jnp.concatenate(
        [ex, ex * gs[:, 0:4], jnp.zeros((EBLK, 3), jnp.float32)], axis=1)


def _sc2_body(p_in, didx2, zeros_in, out, ibuf, pbuf, tbl):
    cid = lax.axis_index("c")
    sid = lax.axis_index("s")
    wid = sid * 2 + cid

    @pl.when(sid == 0)
    def _():
        pltpu.sync_copy(zeros_in, tbl)

    plsc.subcore_barrier()
    rows128 = PER // 128
    pltpu.sync_copy(didx2.at[pl.ds(wid * rows128, rows128)], ibuf)
    pltpu.sync_copy(p_in.at[pl.ds(wid * PER, PER)], pbuf)

    def body(j):
        pltpu.sync_copy(pbuf.at[pl.ds(j * 128, 128)], tbl.at[ibuf.at[j]],
                        add=True)

    pl.loop(0, rows128)(body)
    plsc.subcore_barrier()
    rows = (B * NB) // 16
    pltpu.sync_copy(tbl.at[pl.ds(sid * rows, rows)],
                    out.at[cid, pl.ds(sid * rows, rows)])


def _tc2_body(bins_ref, src2_ref, head_ref, wl_ref, blb_ref, wl2_ref,
              bl2_ref, wr2_ref, br2_ref, att2_ref, ob_ref, vfw_ref,
              vfb_ref, out_ref):
    vals = []
    for b in range(B):
        rows = (bins_ref[0, pl.ds(b * NB, NO), :]
                + bins_ref[1, pl.ds(b * NB, NO), :])
        denom = rows[:, 0:1]
        agg4 = rows[:, 1:5]
        h_obj = jax.nn.relu(
            jnp.dot(agg4 / denom, wl_ref[...],
                    preferred_element_type=jnp.float32) + blb_ref[...])
        hl2 = jnp.dot(h_obj, wl2_ref[...],
                      preferred_element_type=jnp.float32) + bl2_ref[...]
        hr2 = jnp.dot(head_ref[pl.ds(b, 1), :], wr2_ref[...],
                      preferred_element_type=jnp.float32) + br2_ref[...]
        en = jnp.dot(_leaky(hl2 + hr2), att2_ref[...],
                     preferred_element_type=jnp.float32)
        ex2 = jnp.exp(jnp.minimum(en - en[0:1, :], 70.0))
        src2 = src2_ref[b].reshape(E2, 1)
        parts = []
        for j in range(NB // 128):
            band = lax.broadcasted_iota(jnp.int32, (1, 128), 1) + j * 128
            parts.append(jnp.sum((src2 == band).astype(jnp.float32),
                                 axis=0, keepdims=True))
        cnt = jnp.concatenate(parts, axis=1)[:, :NO]
        cnt_t = jnp.transpose(cnt)
        iota_n = lax.broadcasted_iota(jnp.int32, (NO, 1), 0)
        w = jnp.where(iota_n == 0, 1.0, cnt_t * ex2)
        denom2 = jnp.sum(w)
        out_vec = jnp.dot(jnp.transpose(w / denom2), hl2,
                          preferred_element_type=jnp.float32)
        h_head = jax.nn.relu(out_vec + ob_ref[...])
        vals.append(jnp.dot(h_head, vfw_ref[...],
                            preferred_element_type=jnp.float32)
                    + vfb_ref[...])
    out_ref[...] = jnp.concatenate(vals, axis=0)


def kernel(head_node, objective_nodes, value_nodes, ei_vo_src, ei_vo_dst,
           ei_oh_src, ei_oh_dst, opponent_encoding, vo_Wl, vo_bl, vo_Wr,
           vo_br, vo_att, vo_b, oh_Wl, oh_bl, oh_Wr, oh_br, oh_att, oh_b,
           vf_W, vf_b):
    f32 = jnp.float32
    w16s = jnp.concatenate([vo_Wl, jnp.zeros((12, H), f32)], axis=0)
    w16d = jnp.concatenate([vo_Wr, jnp.zeros((14, H), f32)], axis=0)
    b0 = (vo_bl + vo_br).reshape(1, H)
    att = vo_att.reshape(H, 1)
    w16c = jnp.concatenate([vo_Wl, vo_Wr, jnp.zeros((10, H), f32)], axis=0)

    cat0 = jnp.concatenate(
        [value_nodes[:, :NO, :], objective_nodes,
         jnp.zeros((B, NO, 10), f32)], axis=-1)

    c = pl.pallas_call(
        _tc0_body,
        out_shape=jax.ShapeDtypeStruct((B, NO), f32),
    )(cat0, w16c, b0, att)

    src_tab = jnp.concatenate(
        [value_nodes, jnp.zeros((B, NV, 12), f32)], axis=-1
    ).reshape(B * NV, 16)
    dst_tab = jnp.zeros((B, NB, 16), f32)
    dst_tab = dst_tab.at[:, :NO, 0:2].set(objective_nodes)
    dst_tab = dst_tab.at[:, :NO, 2].set(c)
    dst_tab = dst_tab.reshape(B * NB, 16)

    loop = jnp.broadcast_to(jnp.arange(NO, dtype=jnp.int32), (B, NO))
    padz = jnp.zeros((B, EP - EA), jnp.int32)
    src_raw = jnp.concatenate([ei_vo_src, loop, padz], axis=1)
    dst_raw = jnp.concatenate([ei_vo_dst, loop, padz], axis=1)
    boff = jnp.arange(B, dtype=jnp.int32)[:, None]
    src_adj = (src_raw + boff * NV).reshape(NE)
    dst_adj = (dst_raw + boff * NB).reshape(NE)

    mesh = plsc.VectorSubcoreMesh(core_axis_name="c", subcore_axis_name="s")
    sc_params = pltpu.CompilerParams(use_tc_tiling_on_sc=False)
    gs, gd = pl.kernel(
        _sc1_body,
        mesh=mesh,
        compiler_params=sc_params,
        out_type=[jax.ShapeDtypeStruct((NE, 16), f32),
                  jax.ShapeDtypeStruct((NE, 16), f32)],
        scratch_types=[pltpu.VMEM((CH,), jnp.int32),
                       pltpu.VMEM((CH,), jnp.int32),
                       pltpu.VMEM((CH, 16), f32),
                       pltpu.VMEM((CH, 16), f32),
                       pltpu.SemaphoreType.DMA,
                       pltpu.SemaphoreType.DMA],
    )(src_tab, dst_tab, src_adj, dst_adj)

    sr3 = src_raw.reshape(NBLK, 1, EBLK)
    dr3 = dst_raw.reshape(NBLK, 1, EBLK)
    payload = pl.pallas_call(
        _tc1_body,
        grid=(NBLK,),
        in_specs=[
            pl.BlockSpec((EBLK, 16), lambda i: (i, 0)),
            pl.BlockSpec((EBLK, 16), lambda i: (i, 0)),
            pl.BlockSpec((1, 1, EBLK), lambda i: (i, 0, 0)),
            pl.BlockSpec((1, 1, EBLK), lambda i: (i, 0, 0)),
            pl.BlockSpec((16, H), lambda i: (0, 0)),
            pl.BlockSpec((16, H), lambda i: (0, 0)),
            pl.BlockSpec((1, H), lambda i: (0, 0)),
            pl.BlockSpec((H, 1), lambda i: (0, 0)),
        ],
        out_specs=pl.BlockSpec((EBLK, 8), lambda i: (i, 0)),
        out_shape=jax.ShapeDtypeStruct((NE, 8), f32),
    )(gs, gd, sr3, dr3, w16s, w16d, b0, att)

    bins = pl.kernel(
        _sc2_body,
        mesh=mesh,
        compiler_params=sc_params,
        out_type=jax.ShapeDtypeStruct((2, B * NB, 8), f32),
        scratch_types=[pltpu.VMEM((PER // 128, 128), jnp.int32),
                       pltpu.VMEM((PER, 8), f32),
                       pltpu.VMEM_SHARED((B * NB, 8), f32)],
    )(payload, dst_adj.reshape(NE // 128, 128), jnp.zeros((B * NB, 8), f32))

    onehot = jax.nn.one_hot(opponent_encoding[:, 0], NA, dtype=f32)
    head10 = jnp.concatenate([head_node[:, 0, :], onehot], axis=1)

    out2d = pl.pallas_call(
        _tc2_body,
        out_shape=jax.ShapeDtypeStruct((B, 1), f32),
    )(bins, ei_oh_src, head10, vo_Wl, (vo_bl + vo_b).reshape(1, H),
      oh_Wl, oh_bl.reshape(1, H), oh_Wr, oh_br.reshape(1, H),
      oh_att.reshape(H, 1),
      oh_b.reshape(1, H), vf_W, vf_b.reshape(1, 1))
    return out2d[:, 0]
